# baseline (speedup 1.0000x reference)
import functools
import math

import jax
import jax.numpy as jnp
import numpy as np
from jax import lax
from jax.experimental import pallas as pl
from jax.experimental.pallas import tpu as pltpu
from jax.experimental.pallas import tpu_sc as plsc

B, T, C = 32, 384, 256
N_HEAD, N_KV_HEAD = 4, 2
HEAD_DIM = C // N_HEAD
HALF = HEAD_DIM // 2
E = 8
HID = int(8 / 3 * C)
HPAD = 768
N = B * T
TILE = 256
NTOK_TILES = N // TILE
PBUF = N * 2 + E * TILE
NTILES = PBUF // TILE
NW = 32
TPW = N // NW


def _rope_tables():
    inv_freq = 1.0 / (10000.0 ** (np.arange(0, HEAD_DIM, 2).astype(np.float32) / HEAD_DIM))
    t = np.arange(T).astype(np.float32)
    freqs = np.einsum('i,j->ij', t, inv_freq)
    emb = np.concatenate((freqs, freqs), axis=-1)
    return jnp.asarray(np.cos(emb)), jnp.asarray(np.sin(emb))


QB = 128


def _attn_body(x_ref, w1_ref, qwt_ref, kwt_ref, vwt_ref, owt_ref, cos_ref,
               sin_ref, x1_ref):
    xb = x_ref[0]
    xn = xb * lax.rsqrt(jnp.mean(xb * xb, axis=-1, keepdims=True) + 1e-6)
    xn = (xn * w1_ref[...]).astype(jnp.bfloat16)
    q = jnp.dot(xn, qwt_ref[...], preferred_element_type=jnp.float32)
    k = jnp.dot(xn, kwt_ref[...], preferred_element_type=jnp.float32)
    v = jnp.dot(xn, vwt_ref[...], preferred_element_type=jnp.float32)
    cos = cos_ref[...]
    sin = sin_ref[...]

    def rope(u):
        ur = jnp.concatenate([-u[:, HALF:], u[:, :HALF]], axis=1)
        return u * cos + ur * sin

    scale = 1.0 / math.sqrt(HEAD_DIM)
    ys = []
    kh_c = [rope(k[:, h * HEAD_DIM:(h + 1) * HEAD_DIM]).astype(jnp.bfloat16)
            for h in range(N_KV_HEAD)]
    for h in range(N_HEAD):
        qh = rope(q[:, h * HEAD_DIM:(h + 1) * HEAD_DIM]).astype(jnp.bfloat16)
        kh = kh_c[h // 2]
        vh = v[:, (h // 2) * HEAD_DIM:(h // 2 + 1) * HEAD_DIM].astype(jnp.bfloat16)
        blocks = []
        for qb in range(T // QB):
            kl = (qb + 1) * QB
            qhb = qh[qb * QB:(qb + 1) * QB]
            s = lax.dot_general(qhb, kh[:kl], (((1,), (1,)), ((), ())),
                                preferred_element_type=jnp.float32) * scale
            rows = qb * QB + lax.broadcasted_iota(jnp.int32, (QB, kl), 0)
            cols = lax.broadcasted_iota(jnp.int32, (QB, kl), 1)
            s = jnp.where(rows >= cols, s, -1e30)
            m = jnp.max(s, axis=1, keepdims=True)
            p = jnp.exp(s - m)
            a = (p / jnp.sum(p, axis=1, keepdims=True)).astype(jnp.bfloat16)
            blocks.append(jnp.dot(a, vh[:kl], preferred_element_type=jnp.float32))
        ys.append(jnp.concatenate(blocks, axis=0))
    y = jnp.concatenate(ys, axis=1).astype(jnp.bfloat16)
    x1_ref[0] = jnp.dot(y, owt_ref[...], preferred_element_type=jnp.float32) + xb


def _attention(x, ln1_w, qW, kW, vW, oW):
    cos, sin = _rope_tables()
    return pl.pallas_call(
        _attn_body,
        grid=(B,),
        in_specs=[
            pl.BlockSpec((1, T, C), lambda b: (b, 0, 0)),
            pl.BlockSpec((1, C), lambda b: (0, 0)),
            pl.BlockSpec((C, C), lambda b: (0, 0)),
            pl.BlockSpec((C, N_KV_HEAD * HEAD_DIM), lambda b: (0, 0)),
            pl.BlockSpec((C, N_KV_HEAD * HEAD_DIM), lambda b: (0, 0)),
            pl.BlockSpec((C, C), lambda b: (0, 0)),
            pl.BlockSpec((T, HEAD_DIM), lambda b: (0, 0)),
            pl.BlockSpec((T, HEAD_DIM), lambda b: (0, 0)),
        ],
        out_specs=pl.BlockSpec((1, T, C), lambda b: (b, 0, 0)),
        out_shape=jax.ShapeDtypeStruct((B, T, C), jnp.float32),
    )(x, ln1_w.reshape(1, C), qW.T.astype(jnp.bfloat16),
      kW.T.astype(jnp.bfloat16), vW.T.astype(jnp.bfloat16),
      oW.T.astype(jnp.bfloat16), cos, sin)


def _router_body(x_ref, w_ref, rwt_ref, tril_ref, x2_ref, mf_ref, cnt_ref,
                 run_ref):
    t = pl.program_id(0)

    @pl.when(t == 0)
    def _():
        run_ref[...] = jnp.zeros((1, E), jnp.float32)

    xb = x_ref[...]
    xn = xb * lax.rsqrt(jnp.mean(xb * xb, axis=-1, keepdims=True) + 1e-6)
    xn = xn * w_ref[...]
    x2_ref[...] = xn.astype(jnp.bfloat16)

    logits = jnp.dot(xn, rwt_ref[...], preferred_element_type=jnp.float32)
    lm = jnp.max(logits, axis=1, keepdims=True)
    ex = jnp.exp(logits - lm)
    probs = ex / jnp.sum(ex, axis=1, keepdims=True)

    lane = lax.broadcasted_iota(jnp.int32, (TILE, E), 1)
    p0 = jnp.max(probs, axis=1, keepdims=True)
    i0 = jnp.min(jnp.where(probs == p0, lane, E), axis=1, keepdims=True)
    masked = jnp.where(lane == i0, -1.0, probs)
    p1 = jnp.max(masked, axis=1, keepdims=True)
    i1 = jnp.min(jnp.where(masked == p1, lane, E), axis=1, keepdims=True)
    ssum = p0 + p1 + 1e-9
    g0 = p0 / ssum
    g1 = p1 / ssum

    oh0 = (lane == i0).astype(jnp.float32)
    oh1 = (lane == i1).astype(jnp.float32)
    oh = oh0 + oh1
    prefix = jnp.dot(tril_ref[...], oh, preferred_element_type=jnp.float32)
    run = run_ref[...]
    r0 = jnp.sum((prefix + run) * oh0, axis=1, keepdims=True)
    r1 = jnp.sum((prefix + run) * oh1, axis=1, keepdims=True)
    newrun = run + jnp.sum(oh, axis=0, keepdims=True)
    run_ref[...] = newrun
    cnt_ref[...] = newrun

    mf_ref[0] = jnp.concatenate(
        [i0.astype(jnp.float32), i1.astype(jnp.float32), r0, r1, g0, g1,
         jnp.zeros((TILE, 2), jnp.float32)], axis=1)


def _router(x1f, ln2_w, routerW):
    tril = np.tril(np.ones((TILE, TILE), np.float32), k=-1)
    return pl.pallas_call(
        _router_body,
        grid=(NTOK_TILES,),
        in_specs=[
            pl.BlockSpec((TILE, C), lambda t: (t, 0)),
            pl.BlockSpec((1, C), lambda t: (0, 0)),
            pl.BlockSpec((C, E), lambda t: (0, 0)),
            pl.BlockSpec((TILE, TILE), lambda t: (0, 0)),
        ],
        out_specs=[
            pl.BlockSpec((TILE, C), lambda t: (t, 0)),
            pl.BlockSpec((1, TILE, E), lambda t: (t, 0, 0)),
            pl.BlockSpec((1, E), lambda t: (0, 0)),
        ],
        out_shape=[
            jax.ShapeDtypeStruct((N, C), jnp.bfloat16),
            jax.ShapeDtypeStruct((NTOK_TILES, TILE, E), jnp.float32),
            jax.ShapeDtypeStruct((1, E), jnp.float32),
        ],
        scratch_shapes=[pltpu.VMEM((1, E), jnp.float32)],
    )(x1f, ln2_w.reshape(1, C), routerW.T, jnp.asarray(tril))


def _dest_body(mf_ref, cnt_ref, tril8_ref, dd0_ref, dd1_ref, texp_ref):
    cnt = cnt_ref[...]
    padded = jnp.ceil(cnt / TILE) * TILE
    offs = jnp.dot(padded, tril8_ref[...], preferred_element_type=jnp.float32)
    ends = (offs + padded) / TILE
    tt = lax.broadcasted_iota(jnp.int32, (1, 128), 1).astype(jnp.float32)
    acc = jnp.zeros((1, 128), jnp.float32)
    for e in range(E):
        acc = acc + (tt >= ends[0, e]).astype(jnp.float32)
    texp_ref[...] = jnp.minimum(acc, float(E - 1))

    e0 = mf_ref[:, :, 0]
    e1 = mf_ref[:, :, 1]
    d0 = mf_ref[:, :, 2]
    d1 = mf_ref[:, :, 3]
    for e in range(E):
        d0 = d0 + jnp.where(e0 == float(e), offs[0, e], 0.0)
        d1 = d1 + jnp.where(e1 == float(e), offs[0, e], 0.0)
    dd0_ref[...] = d0
    dd1_ref[...] = d1


def _dest(mf, cnt):
    tril8 = np.triu(np.ones((E, E), np.float32), k=1)
    return pl.pallas_call(
        _dest_body,
        out_shape=[
            jax.ShapeDtypeStruct((NTOK_TILES, TILE), jnp.float32),
            jax.ShapeDtypeStruct((NTOK_TILES, TILE), jnp.float32),
            jax.ShapeDtypeStruct((1, 128), jnp.float32),
        ],
    )(mf, cnt, jnp.asarray(tril8))


CH = 128
NCH = TPW // CH


def _dispatch_body(x2_hbm, d0_hbm, d1_hbm, buf_hbm, *refs):
    rows = refs[0:NCH]
    dv0 = refs[NCH:2 * NCH]
    dv1 = refs[2 * NCH:3 * NCH]
    lsem, ssem = refs[3 * NCH], refs[3 * NCH + 1]
    wid = lax.axis_index("s") * 2 + lax.axis_index("c")
    loads = []
    for ci in range(NCH):
        base = wid * TPW + ci * CH
        loads.append(pltpu.async_copy(x2_hbm.at[pl.ds(base, CH)], rows[ci], lsem))
        loads.append(pltpu.async_copy(d0_hbm.at[pl.ds(base, CH)], dv0[ci], lsem))
        loads.append(pltpu.async_copy(d1_hbm.at[pl.ds(base, CH)], dv1[ci], lsem))
    for c in loads:
        c.wait()
    scs = []
    for ci in range(NCH):
        scs.append(pltpu.async_copy(rows[ci], buf_hbm.at[dv0[ci]], ssem))
        scs.append(pltpu.async_copy(rows[ci], buf_hbm.at[dv1[ci]], ssem))
    for c in scs:
        c.wait()


def _dispatch(x2i, d0, d1):
    mesh = plsc.VectorSubcoreMesh(core_axis_name="c", subcore_axis_name="s")
    f = pl.kernel(
        _dispatch_body,
        mesh=mesh,
        out_type=jax.ShapeDtypeStruct((PBUF, C // 2), jnp.int32),
        scratch_types=(
            [pltpu.VMEM((CH, C // 2), jnp.int32) for _ in range(NCH)]
            + [pltpu.VMEM((CH,), jnp.int32) for _ in range(2 * NCH)]
            + [pltpu.SemaphoreType.DMA, pltpu.SemaphoreType.DMA]
        ),
    )
    return f(x2i, d0, d1)


def _gmm_body(texp_ref, xb_ref, w1_ref, w3_ref, w2_ref, y_ref):
    xb = xb_ref[...]
    h1 = lax.dot_general(xb, w1_ref[0], (((1,), (1,)), ((), ())),
                         preferred_element_type=jnp.float32)
    h3 = lax.dot_general(xb, w3_ref[0], (((1,), (1,)), ((), ())),
                         preferred_element_type=jnp.float32)
    h = (h1 * (1.0 / (1.0 + jnp.exp(-h1))) * h3).astype(jnp.bfloat16)
    y = lax.dot_general(h, w2_ref[0], (((1,), (1,)), ((), ())),
                        preferred_element_type=jnp.float32)
    y_ref[...] = y.astype(jnp.bfloat16)


def _grouped_mlp(texp, buf, W1p, W3p, W2p):
    grid_spec = pltpu.PrefetchScalarGridSpec(
        num_scalar_prefetch=1,
        grid=(NTILES,),
        in_specs=[
            pl.BlockSpec((TILE, C), lambda t, s: (t, 0)),
            pl.BlockSpec((1, HPAD, C), lambda t, s: (s[t], 0, 0)),
            pl.BlockSpec((1, HPAD, C), lambda t, s: (s[t], 0, 0)),
            pl.BlockSpec((1, C, HPAD), lambda t, s: (s[t], 0, 0)),
        ],
        out_specs=pl.BlockSpec((TILE, C), lambda t, s: (t, 0)),
    )
    return pl.pallas_call(
        _gmm_body,
        grid_spec=grid_spec,
        out_shape=jax.ShapeDtypeStruct((PBUF, C), jnp.bfloat16),
    )(texp, buf, W1p, W3p, W2p)


CCH = 128


def _gather_body(y_hbm, d0_hbm, d1_hbm, r0_hbm, r1_hbm, *refs):
    ry0 = refs[0:NCH]
    ry1 = refs[NCH:2 * NCH]
    dv0 = refs[2 * NCH:3 * NCH]
    dv1 = refs[3 * NCH:4 * NCH]
    lsem, gsem, ssem = refs[4 * NCH], refs[4 * NCH + 1], refs[4 * NCH + 2]
    wid = lax.axis_index("s") * 2 + lax.axis_index("c")
    loads = []
    for ci in range(NCH):
        base = wid * TPW + ci * CCH
        loads.append(pltpu.async_copy(d0_hbm.at[pl.ds(base, CCH)], dv0[ci], lsem))
        loads.append(pltpu.async_copy(d1_hbm.at[pl.ds(base, CCH)], dv1[ci], lsem))
    for c in loads:
        c.wait()
    gs = []
    for ci in range(NCH):
        gs.append(pltpu.async_copy(y_hbm.at[dv0[ci]], ry0[ci], gsem))
        gs.append(pltpu.async_copy(y_hbm.at[dv1[ci]], ry1[ci], gsem))
    for c in gs:
        c.wait()
    sts = []
    for ci in range(NCH):
        base = wid * TPW + ci * CCH
        sts.append(pltpu.async_copy(ry0[ci], r0_hbm.at[pl.ds(base, CCH)], ssem))
        sts.append(pltpu.async_copy(ry1[ci], r1_hbm.at[pl.ds(base, CCH)], ssem))
    for c in sts:
        c.wait()


def _gather2(yi, d0, d1):
    mesh = plsc.VectorSubcoreMesh(core_axis_name="c", subcore_axis_name="s")
    f = pl.kernel(
        _gather_body,
        mesh=mesh,
        out_type=[
            jax.ShapeDtypeStruct((N, C // 2), jnp.int32),
            jax.ShapeDtypeStruct((N, C // 2), jnp.int32),
        ],
        scratch_types=(
            [pltpu.VMEM((CCH, C // 2), jnp.int32) for _ in range(2 * NCH)]
            + [pltpu.VMEM((CCH,), jnp.int32) for _ in range(2 * NCH)]
            + [pltpu.SemaphoreType.DMA] * 3
        ),
    )
    return f(yi, d0, d1)


def _epilogue_body(x1_ref, ry0_ref, ry1_ref, g0_ref, g1_ref, out_ref):
    out_ref[...] = (x1_ref[...]
                    + g0_ref[...] * ry0_ref[...].astype(jnp.float32)
                    + g1_ref[...] * ry1_ref[...].astype(jnp.float32))


def _epilogue(x1f, ry0, ry1, g0, g1):
    return pl.pallas_call(
        _epilogue_body,
        grid=(NTOK_TILES,),
        in_specs=[
            pl.BlockSpec((TILE, C), lambda t: (t, 0)),
            pl.BlockSpec((TILE, C), lambda t: (t, 0)),
            pl.BlockSpec((TILE, C), lambda t: (t, 0)),
            pl.BlockSpec((TILE, 1), lambda t: (t, 0)),
            pl.BlockSpec((TILE, 1), lambda t: (t, 0)),
        ],
        out_specs=pl.BlockSpec((TILE, C), lambda t: (t, 0)),
        out_shape=jax.ShapeDtypeStruct((N, C), jnp.float32),
    )(x1f, ry0, ry1, g0, g1)


def kernel(x, ln1_w, qW, kW, vW, oW, ln2_w, routerW, W1, W2, W3):
    x1 = _attention(x, ln1_w, qW, kW, vW, oW)
    x1f = x1.reshape(N, C)
    x2, mf, cnt = _router(x1f, ln2_w, routerW)
    dd0, dd1, texpf = _dest(mf, cnt)
    texp = texpf[0, :NTILES].astype(jnp.int32)

    d0 = dd0.reshape(N).astype(jnp.int32)
    d1 = dd1.reshape(N).astype(jnp.int32)
    g0 = mf[:, :, 4].reshape(N, 1)
    g1 = mf[:, :, 5].reshape(N, 1)

    x2i = lax.bitcast_convert_type(x2.reshape(N, C // 2, 2), jnp.int32)
    bufi = _dispatch(x2i, d0, d1)
    buf = lax.bitcast_convert_type(bufi, jnp.bfloat16).reshape(PBUF, C)

    W1p = jnp.pad(W1.astype(jnp.bfloat16), ((0, 0), (0, HPAD - HID), (0, 0)))
    W3p = jnp.pad(W3.astype(jnp.bfloat16), ((0, 0), (0, HPAD - HID), (0, 0)))
    W2p = jnp.pad(W2.astype(jnp.bfloat16), ((0, 0), (0, 0), (0, HPAD - HID)))
    y = _grouped_mlp(texp, buf, W1p, W3p, W2p)

    yi = lax.bitcast_convert_type(y.reshape(PBUF, C // 2, 2), jnp.int32)
    ry0i, ry1i = _gather2(yi, d0, d1)
    ry0 = lax.bitcast_convert_type(ry0i, jnp.bfloat16).reshape(N, C)
    ry1 = lax.bitcast_convert_type(ry1i, jnp.bfloat16).reshape(N, C)
    out = _epilogue(x1f, ry0, ry1, g0, g1)
    return out.reshape(B, T, C)


# speedup vs baseline: 2.0566x; 2.0566x over previous
import functools
import math

import jax
import jax.numpy as jnp
import numpy as np
from jax import lax
from jax.experimental import pallas as pl
from jax.experimental.pallas import tpu as pltpu
from jax.experimental.pallas import tpu_sc as plsc

B, T, C = 32, 384, 256
N_HEAD, N_KV_HEAD = 4, 2
HEAD_DIM = C // N_HEAD
HALF = HEAD_DIM // 2
E = 8
HID = int(8 / 3 * C)
HPAD = 768
N = B * T
TILE = 256
NTOK_TILES = N // TILE
PBUF = N * 2 + E * TILE
NTILES = PBUF // TILE
NW = 32
TPW = N // NW


def _rope_tables():
    inv_freq = 1.0 / (10000.0 ** (np.arange(0, HEAD_DIM, 2).astype(np.float32) / HEAD_DIM))
    t = np.arange(T).astype(np.float32)
    freqs = np.einsum('i,j->ij', t, inv_freq)
    emb = np.concatenate((freqs, freqs), axis=-1)
    return jnp.asarray(np.cos(emb)), jnp.asarray(np.sin(emb))


QB = 128


def _attn_body(x_ref, w1_ref, qwt_ref, kwt_ref, vwt_ref, owt_ref, cos_ref,
               sin_ref, x1_ref):
    xb = x_ref[0]
    xn = xb * lax.rsqrt(jnp.mean(xb * xb, axis=-1, keepdims=True) + 1e-6)
    xn = (xn * w1_ref[...]).astype(jnp.bfloat16)
    q = jnp.dot(xn, qwt_ref[...], preferred_element_type=jnp.float32)
    k = jnp.dot(xn, kwt_ref[...], preferred_element_type=jnp.float32)
    v = jnp.dot(xn, vwt_ref[...], preferred_element_type=jnp.float32)
    cos = cos_ref[...]
    sin = sin_ref[...]

    def rope(u):
        ur = jnp.concatenate([-u[:, HALF:], u[:, :HALF]], axis=1)
        return u * cos + ur * sin

    scale = 1.0 / math.sqrt(HEAD_DIM)
    ys = []
    kh_c = [rope(k[:, h * HEAD_DIM:(h + 1) * HEAD_DIM]).astype(jnp.bfloat16)
            for h in range(N_KV_HEAD)]
    for h in range(N_HEAD):
        qh = rope(q[:, h * HEAD_DIM:(h + 1) * HEAD_DIM]).astype(jnp.bfloat16)
        kh = kh_c[h // 2]
        vh = v[:, (h // 2) * HEAD_DIM:(h // 2 + 1) * HEAD_DIM].astype(jnp.bfloat16)
        blocks = []
        for qb in range(T // QB):
            kl = (qb + 1) * QB
            qhb = qh[qb * QB:(qb + 1) * QB]
            s = lax.dot_general(qhb, kh[:kl], (((1,), (1,)), ((), ())),
                                preferred_element_type=jnp.float32) * scale
            rows = qb * QB + lax.broadcasted_iota(jnp.int32, (QB, kl), 0)
            cols = lax.broadcasted_iota(jnp.int32, (QB, kl), 1)
            s = jnp.where(rows >= cols, s, -1e30)
            m = jnp.max(s, axis=1, keepdims=True)
            p = jnp.exp(s - m)
            a = (p / jnp.sum(p, axis=1, keepdims=True)).astype(jnp.bfloat16)
            blocks.append(jnp.dot(a, vh[:kl], preferred_element_type=jnp.float32))
        ys.append(jnp.concatenate(blocks, axis=0))
    y = jnp.concatenate(ys, axis=1).astype(jnp.bfloat16)
    x1_ref[0] = jnp.dot(y, owt_ref[...], preferred_element_type=jnp.float32) + xb


def _attention(x, ln1_w, qW, kW, vW, oW):
    cos, sin = _rope_tables()
    return pl.pallas_call(
        _attn_body,
        grid=(B,),
        in_specs=[
            pl.BlockSpec((1, T, C), lambda b: (b, 0, 0)),
            pl.BlockSpec((1, C), lambda b: (0, 0)),
            pl.BlockSpec((C, C), lambda b: (0, 0)),
            pl.BlockSpec((C, N_KV_HEAD * HEAD_DIM), lambda b: (0, 0)),
            pl.BlockSpec((C, N_KV_HEAD * HEAD_DIM), lambda b: (0, 0)),
            pl.BlockSpec((C, C), lambda b: (0, 0)),
            pl.BlockSpec((T, HEAD_DIM), lambda b: (0, 0)),
            pl.BlockSpec((T, HEAD_DIM), lambda b: (0, 0)),
        ],
        out_specs=pl.BlockSpec((1, T, C), lambda b: (b, 0, 0)),
        out_shape=jax.ShapeDtypeStruct((B, T, C), jnp.float32),
    )(x, ln1_w.reshape(1, C), qW.T.astype(jnp.bfloat16),
      kW.T.astype(jnp.bfloat16), vW.T.astype(jnp.bfloat16),
      oW.T.astype(jnp.bfloat16), cos, sin)


def _router_body(x_ref, w_ref, rwt_ref, tril_ref, x2_ref, mf_ref, cnt_ref,
                 run_ref):
    t = pl.program_id(0)

    @pl.when(t == 0)
    def _():
        run_ref[...] = jnp.zeros((1, E), jnp.float32)

    xb = x_ref[...]
    xn = xb * lax.rsqrt(jnp.mean(xb * xb, axis=-1, keepdims=True) + 1e-6)
    xn = xn * w_ref[...]
    x2_ref[...] = xn

    logits = jnp.dot(xn, rwt_ref[...], preferred_element_type=jnp.float32)
    lm = jnp.max(logits, axis=1, keepdims=True)
    ex = jnp.exp(logits - lm)
    probs = ex / jnp.sum(ex, axis=1, keepdims=True)

    lane = lax.broadcasted_iota(jnp.int32, (TILE, E), 1)
    p0 = jnp.max(probs, axis=1, keepdims=True)
    i0 = jnp.min(jnp.where(probs == p0, lane, E), axis=1, keepdims=True)
    masked = jnp.where(lane == i0, -1.0, probs)
    p1 = jnp.max(masked, axis=1, keepdims=True)
    i1 = jnp.min(jnp.where(masked == p1, lane, E), axis=1, keepdims=True)
    ssum = p0 + p1 + 1e-9
    g0 = p0 / ssum
    g1 = p1 / ssum

    oh0 = (lane == i0).astype(jnp.float32)
    oh1 = (lane == i1).astype(jnp.float32)
    oh = oh0 + oh1
    prefix = jnp.dot(tril_ref[...], oh, preferred_element_type=jnp.float32)
    run = run_ref[...]
    r0 = jnp.sum((prefix + run) * oh0, axis=1, keepdims=True)
    r1 = jnp.sum((prefix + run) * oh1, axis=1, keepdims=True)
    newrun = run + jnp.sum(oh, axis=0, keepdims=True)
    run_ref[...] = newrun
    cnt_ref[...] = newrun

    mf_ref[0] = jnp.concatenate(
        [i0.astype(jnp.float32), i1.astype(jnp.float32), r0, r1, g0, g1,
         jnp.zeros((TILE, 2), jnp.float32)], axis=1)


def _router(x1f, ln2_w, routerW):
    tril = np.tril(np.ones((TILE, TILE), np.float32), k=-1)
    return pl.pallas_call(
        _router_body,
        grid=(NTOK_TILES,),
        in_specs=[
            pl.BlockSpec((TILE, C), lambda t: (t, 0)),
            pl.BlockSpec((1, C), lambda t: (0, 0)),
            pl.BlockSpec((C, E), lambda t: (0, 0)),
            pl.BlockSpec((TILE, TILE), lambda t: (0, 0)),
        ],
        out_specs=[
            pl.BlockSpec((TILE, C), lambda t: (t, 0)),
            pl.BlockSpec((1, TILE, E), lambda t: (t, 0, 0)),
            pl.BlockSpec((1, E), lambda t: (0, 0)),
        ],
        out_shape=[
            jax.ShapeDtypeStruct((N, C), jnp.float32),
            jax.ShapeDtypeStruct((NTOK_TILES, TILE, E), jnp.float32),
            jax.ShapeDtypeStruct((1, E), jnp.float32),
        ],
        scratch_shapes=[pltpu.VMEM((1, E), jnp.float32)],
    )(x1f, ln2_w.reshape(1, C), routerW.T, jnp.asarray(tril))


def _dest_body(mf_ref, cnt_ref, tril8_ref, dd0_ref, dd1_ref, texp_ref):
    cnt = cnt_ref[...]
    padded = jnp.ceil(cnt / TILE) * TILE
    offs = jnp.dot(padded, tril8_ref[...], preferred_element_type=jnp.float32)
    ends = (offs + padded) / TILE
    tt = lax.broadcasted_iota(jnp.int32, (1, 128), 1).astype(jnp.float32)
    acc = jnp.zeros((1, 128), jnp.float32)
    for e in range(E):
        acc = acc + (tt >= ends[0, e]).astype(jnp.float32)
    texp_ref[...] = jnp.minimum(acc, float(E - 1))

    e0 = mf_ref[:, :, 0]
    e1 = mf_ref[:, :, 1]
    d0 = mf_ref[:, :, 2]
    d1 = mf_ref[:, :, 3]
    for e in range(E):
        d0 = d0 + jnp.where(e0 == float(e), offs[0, e], 0.0)
        d1 = d1 + jnp.where(e1 == float(e), offs[0, e], 0.0)
    dd0_ref[...] = d0
    dd1_ref[...] = d1


def _dest(mf, cnt):
    tril8 = np.triu(np.ones((E, E), np.float32), k=1)
    return pl.pallas_call(
        _dest_body,
        out_shape=[
            jax.ShapeDtypeStruct((NTOK_TILES, TILE), jnp.float32),
            jax.ShapeDtypeStruct((NTOK_TILES, TILE), jnp.float32),
            jax.ShapeDtypeStruct((1, 128), jnp.float32),
        ],
    )(mf, cnt, jnp.asarray(tril8))


CH = 128
NCH = TPW // CH


def _dispatch_body(x2_hbm, d0_hbm, d1_hbm, buf_hbm, *refs):
    rows = refs[0:NCH]
    dv0 = refs[NCH:2 * NCH]
    dv1 = refs[2 * NCH:3 * NCH]
    lsem, ssem = refs[3 * NCH], refs[3 * NCH + 1]
    wid = lax.axis_index("s") * 2 + lax.axis_index("c")
    loads = []
    for ci in range(NCH):
        base = wid * TPW + ci * CH
        loads.append(pltpu.async_copy(x2_hbm.at[pl.ds(base, CH)], rows[ci], lsem))
        loads.append(pltpu.async_copy(d0_hbm.at[pl.ds(base, CH)], dv0[ci], lsem))
        loads.append(pltpu.async_copy(d1_hbm.at[pl.ds(base, CH)], dv1[ci], lsem))
    for c in loads:
        c.wait()
    scs = []
    for ci in range(NCH):
        scs.append(pltpu.async_copy(rows[ci], buf_hbm.at[dv0[ci]], ssem))
        scs.append(pltpu.async_copy(rows[ci], buf_hbm.at[dv1[ci]], ssem))
    for c in scs:
        c.wait()


def _dispatch(x2, d0, d1):
    mesh = plsc.VectorSubcoreMesh(core_axis_name="c", subcore_axis_name="s")
    f = pl.kernel(
        _dispatch_body,
        mesh=mesh,
        out_type=jax.ShapeDtypeStruct((PBUF, C), jnp.float32),
        scratch_types=(
            [pltpu.VMEM((CH, C), jnp.float32) for _ in range(NCH)]
            + [pltpu.VMEM((CH,), jnp.int32) for _ in range(2 * NCH)]
            + [pltpu.SemaphoreType.DMA, pltpu.SemaphoreType.DMA]
        ),
    )
    return f(x2, d0, d1)


def _gmm_body(texp_ref, xb_ref, w1_ref, w3_ref, w2_ref, y_ref):
    xb = xb_ref[...].astype(jnp.bfloat16)
    h1 = lax.dot_general(xb, w1_ref[0], (((1,), (1,)), ((), ())),
                         preferred_element_type=jnp.float32)
    h3 = lax.dot_general(xb, w3_ref[0], (((1,), (1,)), ((), ())),
                         preferred_element_type=jnp.float32)
    h = (h1 * (1.0 / (1.0 + jnp.exp(-h1))) * h3).astype(jnp.bfloat16)
    y_ref[...] = lax.dot_general(h, w2_ref[0], (((1,), (1,)), ((), ())),
                                 preferred_element_type=jnp.float32)


def _grouped_mlp(texp, buf, W1p, W3p, W2p):
    grid_spec = pltpu.PrefetchScalarGridSpec(
        num_scalar_prefetch=1,
        grid=(NTILES,),
        in_specs=[
            pl.BlockSpec((TILE, C), lambda t, s: (t, 0)),
            pl.BlockSpec((1, HPAD, C), lambda t, s: (s[t], 0, 0)),
            pl.BlockSpec((1, HPAD, C), lambda t, s: (s[t], 0, 0)),
            pl.BlockSpec((1, C, HPAD), lambda t, s: (s[t], 0, 0)),
        ],
        out_specs=pl.BlockSpec((TILE, C), lambda t, s: (t, 0)),
    )
    return pl.pallas_call(
        _gmm_body,
        grid_spec=grid_spec,
        out_shape=jax.ShapeDtypeStruct((PBUF, C), jnp.float32),
    )(texp, buf, W1p, W3p, W2p)


CCH = 64
NCHG = TPW // CCH


def _gather_body(y_hbm, d0_hbm, d1_hbm, r0_hbm, r1_hbm, *refs):
    ry0 = refs[0:2]
    ry1 = refs[2:4]
    dv0 = refs[4:6]
    dv1 = refs[6:8]
    gsem, ssem = refs[8], refs[9]
    wid = lax.axis_index("s") * 2 + lax.axis_index("c")
    stores = [None, None]
    for ci in range(NCHG):
        s = ci % 2
        if stores[s] is not None:
            stores[s][0].wait()
            stores[s][1].wait()
        base = wid * TPW + ci * CCH
        pltpu.sync_copy(d0_hbm.at[pl.ds(base, CCH)], dv0[s])
        pltpu.sync_copy(d1_hbm.at[pl.ds(base, CCH)], dv1[s])
        c0 = pltpu.async_copy(y_hbm.at[dv0[s]], ry0[s], gsem)
        c1 = pltpu.async_copy(y_hbm.at[dv1[s]], ry1[s], gsem)
        c0.wait()
        c1.wait()
        stores[s] = (
            pltpu.async_copy(ry0[s], r0_hbm.at[pl.ds(base, CCH)], ssem),
            pltpu.async_copy(ry1[s], r1_hbm.at[pl.ds(base, CCH)], ssem),
        )
    for st in stores:
        st[0].wait()
        st[1].wait()


def _gather2(y, d0, d1):
    mesh = plsc.VectorSubcoreMesh(core_axis_name="c", subcore_axis_name="s")
    f = pl.kernel(
        _gather_body,
        mesh=mesh,
        out_type=[
            jax.ShapeDtypeStruct((N, C), jnp.float32),
            jax.ShapeDtypeStruct((N, C), jnp.float32),
        ],
        scratch_types=(
            [pltpu.VMEM((CCH, C), jnp.float32) for _ in range(4)]
            + [pltpu.VMEM((CCH,), jnp.int32) for _ in range(4)]
            + [pltpu.SemaphoreType.DMA] * 2
        ),
    )
    return f(y, d0, d1)


def _epilogue_body(x1_ref, ry0_ref, ry1_ref, g0_ref, g1_ref, out_ref):
    out_ref[...] = (x1_ref[...] + g0_ref[...] * ry0_ref[...]
                    + g1_ref[...] * ry1_ref[...])


def _epilogue(x1f, ry0, ry1, g0, g1):
    return pl.pallas_call(
        _epilogue_body,
        grid=(NTOK_TILES,),
        in_specs=[
            pl.BlockSpec((TILE, C), lambda t: (t, 0)),
            pl.BlockSpec((TILE, C), lambda t: (t, 0)),
            pl.BlockSpec((TILE, C), lambda t: (t, 0)),
            pl.BlockSpec((TILE, 1), lambda t: (t, 0)),
            pl.BlockSpec((TILE, 1), lambda t: (t, 0)),
        ],
        out_specs=pl.BlockSpec((TILE, C), lambda t: (t, 0)),
        out_shape=jax.ShapeDtypeStruct((N, C), jnp.float32),
    )(x1f, ry0, ry1, g0, g1)


def kernel(x, ln1_w, qW, kW, vW, oW, ln2_w, routerW, W1, W2, W3):
    x1 = _attention(x, ln1_w, qW, kW, vW, oW)
    x1f = x1.reshape(N, C)
    x2, mf, cnt = _router(x1f, ln2_w, routerW)
    dd0, dd1, texpf = _dest(mf, cnt)
    texp = texpf[0, :NTILES].astype(jnp.int32)

    d0 = dd0.reshape(N).astype(jnp.int32)
    d1 = dd1.reshape(N).astype(jnp.int32)
    g0 = mf[:, :, 4].reshape(N, 1)
    g1 = mf[:, :, 5].reshape(N, 1)

    buf = _dispatch(x2, d0, d1)

    W1p = jnp.pad(W1.astype(jnp.bfloat16), ((0, 0), (0, HPAD - HID), (0, 0)))
    W3p = jnp.pad(W3.astype(jnp.bfloat16), ((0, 0), (0, HPAD - HID), (0, 0)))
    W2p = jnp.pad(W2.astype(jnp.bfloat16), ((0, 0), (0, 0), (0, HPAD - HID)))
    y = _grouped_mlp(texp, buf, W1p, W3p, W2p)

    ry0, ry1 = _gather2(y, d0, d1)
    out = _epilogue(x1f, ry0, ry1, g0, g1)
    return out.reshape(B, T, C)


# speedup vs baseline: 2.2582x; 1.0980x over previous
import functools
import math

import jax
import jax.numpy as jnp
import numpy as np
from jax import lax
from jax.experimental import pallas as pl
from jax.experimental.pallas import tpu as pltpu
from jax.experimental.pallas import tpu_sc as plsc

B, T, C = 32, 384, 256
N_HEAD, N_KV_HEAD = 4, 2
HEAD_DIM = C // N_HEAD
HALF = HEAD_DIM // 2
E = 8
HID = int(8 / 3 * C)
HPAD = 768
N = B * T
TILE = 256
NTOK_TILES = N // TILE
PBUF = N * 2 + E * TILE
NTILES = PBUF // TILE
NW = 32
TPW = N // NW


def _rope_tables():
    inv_freq = 1.0 / (10000.0 ** (np.arange(0, HEAD_DIM, 2).astype(np.float32) / HEAD_DIM))
    t = np.arange(T).astype(np.float32)
    freqs = np.einsum('i,j->ij', t, inv_freq)
    emb = np.concatenate((freqs, freqs), axis=-1)
    return jnp.asarray(np.cos(emb)), jnp.asarray(np.sin(emb))


QB = 128


def _attn_body(x_ref, w1_ref, qwt_ref, kwt_ref, vwt_ref, owt_ref, cos_ref,
               sin_ref, x1_ref):
    xb = x_ref[0]
    xn = xb * lax.rsqrt(jnp.mean(xb * xb, axis=-1, keepdims=True) + 1e-6)
    xn = (xn * w1_ref[...]).astype(jnp.bfloat16)
    q = jnp.dot(xn, qwt_ref[...], preferred_element_type=jnp.float32)
    k = jnp.dot(xn, kwt_ref[...], preferred_element_type=jnp.float32)
    v = jnp.dot(xn, vwt_ref[...], preferred_element_type=jnp.float32)
    cos = cos_ref[...]
    sin = sin_ref[...]

    def rope(u):
        ur = jnp.concatenate([-u[:, HALF:], u[:, :HALF]], axis=1)
        return u * cos + ur * sin

    rows = lax.broadcasted_iota(jnp.int32, (T, T), 0)
    cols = lax.broadcasted_iota(jnp.int32, (T, T), 1)
    causal = rows >= cols
    scale = 1.0 / math.sqrt(HEAD_DIM)
    ys = []
    kh_c = [rope(k[:, h * HEAD_DIM:(h + 1) * HEAD_DIM]).astype(jnp.bfloat16)
            for h in range(N_KV_HEAD)]
    for h in range(N_HEAD):
        qh = rope(q[:, h * HEAD_DIM:(h + 1) * HEAD_DIM]).astype(jnp.bfloat16)
        kh = kh_c[h // 2]
        vh = v[:, (h // 2) * HEAD_DIM:(h // 2 + 1) * HEAD_DIM].astype(jnp.bfloat16)
        s = lax.dot_general(qh, kh, (((1,), (1,)), ((), ())),
                            preferred_element_type=jnp.float32) * scale
        s = jnp.where(causal, s, -1e30)
        m = jnp.max(s, axis=1, keepdims=True)
        p = jnp.exp(s - m)
        a = (p / jnp.sum(p, axis=1, keepdims=True)).astype(jnp.bfloat16)
        ys.append(jnp.dot(a, vh, preferred_element_type=jnp.float32))
    y = jnp.concatenate(ys, axis=1).astype(jnp.bfloat16)
    x1_ref[0] = jnp.dot(y, owt_ref[...], preferred_element_type=jnp.float32) + xb


def _attention(x, ln1_w, qW, kW, vW, oW):
    cos, sin = _rope_tables()
    return pl.pallas_call(
        _attn_body,
        grid=(B,),
        in_specs=[
            pl.BlockSpec((1, T, C), lambda b: (b, 0, 0)),
            pl.BlockSpec((1, C), lambda b: (0, 0)),
            pl.BlockSpec((C, C), lambda b: (0, 0)),
            pl.BlockSpec((C, N_KV_HEAD * HEAD_DIM), lambda b: (0, 0)),
            pl.BlockSpec((C, N_KV_HEAD * HEAD_DIM), lambda b: (0, 0)),
            pl.BlockSpec((C, C), lambda b: (0, 0)),
            pl.BlockSpec((T, HEAD_DIM), lambda b: (0, 0)),
            pl.BlockSpec((T, HEAD_DIM), lambda b: (0, 0)),
        ],
        out_specs=pl.BlockSpec((1, T, C), lambda b: (b, 0, 0)),
        out_shape=jax.ShapeDtypeStruct((B, T, C), jnp.float32),
    )(x, ln1_w.reshape(1, C), qW.T.astype(jnp.bfloat16),
      kW.T.astype(jnp.bfloat16), vW.T.astype(jnp.bfloat16),
      oW.T.astype(jnp.bfloat16), cos, sin)


def _router_body(x_ref, w_ref, rwt_ref, tril_ref, x2_ref, mf_ref, cnt_ref,
                 run_ref):
    t = pl.program_id(0)

    @pl.when(t == 0)
    def _():
        run_ref[...] = jnp.zeros((1, E), jnp.float32)

    xb = x_ref[...]
    xn = xb * lax.rsqrt(jnp.mean(xb * xb, axis=-1, keepdims=True) + 1e-6)
    xn = xn * w_ref[...]
    x2_ref[...] = xn

    logits = jnp.dot(xn, rwt_ref[...], preferred_element_type=jnp.float32)
    lm = jnp.max(logits, axis=1, keepdims=True)
    ex = jnp.exp(logits - lm)
    probs = ex / jnp.sum(ex, axis=1, keepdims=True)

    lane = lax.broadcasted_iota(jnp.int32, (TILE, E), 1)
    p0 = jnp.max(probs, axis=1, keepdims=True)
    i0 = jnp.min(jnp.where(probs == p0, lane, E), axis=1, keepdims=True)
    masked = jnp.where(lane == i0, -1.0, probs)
    p1 = jnp.max(masked, axis=1, keepdims=True)
    i1 = jnp.min(jnp.where(masked == p1, lane, E), axis=1, keepdims=True)
    ssum = p0 + p1 + 1e-9
    g0 = p0 / ssum
    g1 = p1 / ssum

    oh0 = (lane == i0).astype(jnp.float32)
    oh1 = (lane == i1).astype(jnp.float32)
    oh = oh0 + oh1
    prefix = jnp.dot(tril_ref[...], oh, preferred_element_type=jnp.float32)
    run = run_ref[...]
    r0 = jnp.sum((prefix + run) * oh0, axis=1, keepdims=True)
    r1 = jnp.sum((prefix + run) * oh1, axis=1, keepdims=True)
    newrun = run + jnp.sum(oh, axis=0, keepdims=True)
    run_ref[...] = newrun
    cnt_ref[...] = newrun

    mf_ref[0] = jnp.concatenate(
        [i0.astype(jnp.float32), i1.astype(jnp.float32), r0, r1, g0, g1,
         jnp.zeros((TILE, 2), jnp.float32)], axis=1)


def _router(x1f, ln2_w, routerW):
    tril = np.tril(np.ones((TILE, TILE), np.float32), k=-1)
    return pl.pallas_call(
        _router_body,
        grid=(NTOK_TILES,),
        in_specs=[
            pl.BlockSpec((TILE, C), lambda t: (t, 0)),
            pl.BlockSpec((1, C), lambda t: (0, 0)),
            pl.BlockSpec((C, E), lambda t: (0, 0)),
            pl.BlockSpec((TILE, TILE), lambda t: (0, 0)),
        ],
        out_specs=[
            pl.BlockSpec((TILE, C), lambda t: (t, 0)),
            pl.BlockSpec((1, TILE, E), lambda t: (t, 0, 0)),
            pl.BlockSpec((1, E), lambda t: (0, 0)),
        ],
        out_shape=[
            jax.ShapeDtypeStruct((N, C), jnp.float32),
            jax.ShapeDtypeStruct((NTOK_TILES, TILE, E), jnp.float32),
            jax.ShapeDtypeStruct((1, E), jnp.float32),
        ],
        scratch_shapes=[pltpu.VMEM((1, E), jnp.float32)],
    )(x1f, ln2_w.reshape(1, C), routerW.T, jnp.asarray(tril))


def _dest_body(mf_ref, cnt_ref, tril8_ref, dd0_ref, dd1_ref, texp_ref):
    cnt = cnt_ref[...]
    padded = jnp.ceil(cnt / TILE) * TILE
    offs = jnp.dot(padded, tril8_ref[...], preferred_element_type=jnp.float32)
    ends = (offs + padded) / TILE
    tt = lax.broadcasted_iota(jnp.int32, (1, 128), 1).astype(jnp.float32)
    acc = jnp.zeros((1, 128), jnp.float32)
    for e in range(E):
        acc = acc + (tt >= ends[0, e]).astype(jnp.float32)
    texp_ref[...] = jnp.minimum(acc, float(E - 1))

    e0 = mf_ref[:, :, 0]
    e1 = mf_ref[:, :, 1]
    d0 = mf_ref[:, :, 2]
    d1 = mf_ref[:, :, 3]
    for e in range(E):
        d0 = d0 + jnp.where(e0 == float(e), offs[0, e], 0.0)
        d1 = d1 + jnp.where(e1 == float(e), offs[0, e], 0.0)
    dd0_ref[...] = d0
    dd1_ref[...] = d1


def _dest(mf, cnt):
    tril8 = np.triu(np.ones((E, E), np.float32), k=1)
    return pl.pallas_call(
        _dest_body,
        out_shape=[
            jax.ShapeDtypeStruct((NTOK_TILES, TILE), jnp.float32),
            jax.ShapeDtypeStruct((NTOK_TILES, TILE), jnp.float32),
            jax.ShapeDtypeStruct((1, 128), jnp.float32),
        ],
    )(mf, cnt, jnp.asarray(tril8))


CH = 128
NCH = TPW // CH


def _dispatch_body(x2_hbm, d0_hbm, d1_hbm, buf_hbm, *refs):
    rows = refs[0:NCH]
    dv0 = refs[NCH:2 * NCH]
    dv1 = refs[2 * NCH:3 * NCH]
    lsem, ssem = refs[3 * NCH], refs[3 * NCH + 1]
    wid = lax.axis_index("s") * 2 + lax.axis_index("c")
    loads = []
    for ci in range(NCH):
        base = wid * TPW + ci * CH
        loads.append(pltpu.async_copy(x2_hbm.at[pl.ds(base, CH)], rows[ci], lsem))
        loads.append(pltpu.async_copy(d0_hbm.at[pl.ds(base, CH)], dv0[ci], lsem))
        loads.append(pltpu.async_copy(d1_hbm.at[pl.ds(base, CH)], dv1[ci], lsem))
    for c in loads:
        c.wait()
    scs = []
    for ci in range(NCH):
        scs.append(pltpu.async_copy(rows[ci], buf_hbm.at[dv0[ci]], ssem))
        scs.append(pltpu.async_copy(rows[ci], buf_hbm.at[dv1[ci]], ssem))
    for c in scs:
        c.wait()


def _dispatch(x2, d0, d1):
    mesh = plsc.VectorSubcoreMesh(core_axis_name="c", subcore_axis_name="s")
    f = pl.kernel(
        _dispatch_body,
        mesh=mesh,
        out_type=jax.ShapeDtypeStruct((PBUF, C), jnp.float32),
        scratch_types=(
            [pltpu.VMEM((CH, C), jnp.float32) for _ in range(NCH)]
            + [pltpu.VMEM((CH,), jnp.int32) for _ in range(2 * NCH)]
            + [pltpu.SemaphoreType.DMA, pltpu.SemaphoreType.DMA]
        ),
    )
    return f(x2, d0, d1)


def _gmm_body(texp_ref, xb_ref, w1_ref, w3_ref, w2_ref, y_ref):
    xb = xb_ref[...].astype(jnp.bfloat16)
    h1 = lax.dot_general(xb, w1_ref[0], (((1,), (1,)), ((), ())),
                         preferred_element_type=jnp.float32)
    h3 = lax.dot_general(xb, w3_ref[0], (((1,), (1,)), ((), ())),
                         preferred_element_type=jnp.float32)
    h = (h1 * (1.0 / (1.0 + jnp.exp(-h1))) * h3).astype(jnp.bfloat16)
    y_ref[...] = lax.dot_general(h, w2_ref[0], (((1,), (1,)), ((), ())),
                                 preferred_element_type=jnp.float32)


def _grouped_mlp(texp, buf, W1p, W3p, W2p):
    grid_spec = pltpu.PrefetchScalarGridSpec(
        num_scalar_prefetch=1,
        grid=(NTILES,),
        in_specs=[
            pl.BlockSpec((TILE, C), lambda t, s: (t, 0)),
            pl.BlockSpec((1, HPAD, C), lambda t, s: (s[t], 0, 0)),
            pl.BlockSpec((1, HPAD, C), lambda t, s: (s[t], 0, 0)),
            pl.BlockSpec((1, C, HPAD), lambda t, s: (s[t], 0, 0)),
        ],
        out_specs=pl.BlockSpec((TILE, C), lambda t, s: (t, 0)),
    )
    return pl.pallas_call(
        _gmm_body,
        grid_spec=grid_spec,
        out_shape=jax.ShapeDtypeStruct((PBUF, C), jnp.float32),
    )(texp, buf, W1p, W3p, W2p)


CCH = 64
NCHG = TPW // CCH


def _gather_body(y_hbm, d0_hbm, d1_hbm, r0_hbm, r1_hbm, *refs):
    ry0 = refs[0:2]
    ry1 = refs[2:4]
    dv0 = refs[4:6]
    dv1 = refs[6:8]
    gsem, ssem = refs[8], refs[9]
    wid = lax.axis_index("s") * 2 + lax.axis_index("c")
    stores = [None, None]
    for ci in range(NCHG):
        s = ci % 2
        if stores[s] is not None:
            stores[s][0].wait()
            stores[s][1].wait()
        base = wid * TPW + ci * CCH
        pltpu.sync_copy(d0_hbm.at[pl.ds(base, CCH)], dv0[s])
        pltpu.sync_copy(d1_hbm.at[pl.ds(base, CCH)], dv1[s])
        c0 = pltpu.async_copy(y_hbm.at[dv0[s]], ry0[s], gsem)
        c1 = pltpu.async_copy(y_hbm.at[dv1[s]], ry1[s], gsem)
        c0.wait()
        c1.wait()
        stores[s] = (
            pltpu.async_copy(ry0[s], r0_hbm.at[pl.ds(base, CCH)], ssem),
            pltpu.async_copy(ry1[s], r1_hbm.at[pl.ds(base, CCH)], ssem),
        )
    for st in stores:
        st[0].wait()
        st[1].wait()


def _gather2(y, d0, d1):
    mesh = plsc.VectorSubcoreMesh(core_axis_name="c", subcore_axis_name="s")
    f = pl.kernel(
        _gather_body,
        mesh=mesh,
        out_type=[
            jax.ShapeDtypeStruct((N, C), jnp.float32),
            jax.ShapeDtypeStruct((N, C), jnp.float32),
        ],
        scratch_types=(
            [pltpu.VMEM((CCH, C), jnp.float32) for _ in range(4)]
            + [pltpu.VMEM((CCH,), jnp.int32) for _ in range(4)]
            + [pltpu.SemaphoreType.DMA] * 2
        ),
    )
    return f(y, d0, d1)


def _epilogue_body(x1_ref, ry0_ref, ry1_ref, g0_ref, g1_ref, out_ref):
    out_ref[...] = (x1_ref[...] + g0_ref[...] * ry0_ref[...]
                    + g1_ref[...] * ry1_ref[...])


def _epilogue(x1f, ry0, ry1, g0, g1):
    return pl.pallas_call(
        _epilogue_body,
        grid=(NTOK_TILES,),
        in_specs=[
            pl.BlockSpec((TILE, C), lambda t: (t, 0)),
            pl.BlockSpec((TILE, C), lambda t: (t, 0)),
            pl.BlockSpec((TILE, C), lambda t: (t, 0)),
            pl.BlockSpec((TILE, 1), lambda t: (t, 0)),
            pl.BlockSpec((TILE, 1), lambda t: (t, 0)),
        ],
        out_specs=pl.BlockSpec((TILE, C), lambda t: (t, 0)),
        out_shape=jax.ShapeDtypeStruct((N, C), jnp.float32),
    )(x1f, ry0, ry1, g0, g1)


def kernel(x, ln1_w, qW, kW, vW, oW, ln2_w, routerW, W1, W2, W3):
    x1 = _attention(x, ln1_w, qW, kW, vW, oW)
    x1f = x1.reshape(N, C)
    x2, mf, cnt = _router(x1f, ln2_w, routerW)
    dd0, dd1, texpf = _dest(mf, cnt)
    texp = texpf[0, :NTILES].astype(jnp.int32)

    d0 = dd0.reshape(N).astype(jnp.int32)
    d1 = dd1.reshape(N).astype(jnp.int32)
    g0 = mf[:, :, 4].reshape(N, 1)
    g1 = mf[:, :, 5].reshape(N, 1)

    buf = _dispatch(x2, d0, d1)

    W1p = jnp.pad(W1.astype(jnp.bfloat16), ((0, 0), (0, HPAD - HID), (0, 0)))
    W3p = jnp.pad(W3.astype(jnp.bfloat16), ((0, 0), (0, HPAD - HID), (0, 0)))
    W2p = jnp.pad(W2.astype(jnp.bfloat16), ((0, 0), (0, 0), (0, HPAD - HID)))
    y = _grouped_mlp(texp, buf, W1p, W3p, W2p)

    ry0, ry1 = _gather2(y, d0, d1)
    out = _epilogue(x1f, ry0, ry1, g0, g1)
    return out.reshape(B, T, C)


# speedup vs baseline: 2.2599x; 1.0008x over previous
import functools
import math

import jax
import jax.numpy as jnp
import numpy as np
from jax import lax
from jax.experimental import pallas as pl
from jax.experimental.pallas import tpu as pltpu
from jax.experimental.pallas import tpu_sc as plsc

B, T, C = 32, 384, 256
N_HEAD, N_KV_HEAD = 4, 2
HEAD_DIM = C // N_HEAD
HALF = HEAD_DIM // 2
E = 8
HID = int(8 / 3 * C)
HPAD = 768
N = B * T
TILE = 256
NTOK_TILES = N // TILE
PBUF = N * 2 + E * TILE
NTILES = PBUF // TILE
NW = 32
TPW = N // NW


def _rope_tables():
    inv_freq = 1.0 / (10000.0 ** (np.arange(0, HEAD_DIM, 2).astype(np.float32) / HEAD_DIM))
    t = np.arange(T).astype(np.float32)
    freqs = np.einsum('i,j->ij', t, inv_freq)
    emb = np.concatenate((freqs, freqs), axis=-1)
    return jnp.asarray(np.cos(emb)), jnp.asarray(np.sin(emb))


QB = 128


BB = 2


def _attn_one(xb, w1_ref, qwt_ref, kwt_ref, vwt_ref, owt_ref, cos_ref,
              sin_ref):
    xn = xb * lax.rsqrt(jnp.mean(xb * xb, axis=-1, keepdims=True) + 1e-6)
    xn = (xn * w1_ref[...]).astype(jnp.bfloat16)
    q = jnp.dot(xn, qwt_ref[...], preferred_element_type=jnp.float32)
    k = jnp.dot(xn, kwt_ref[...], preferred_element_type=jnp.float32)
    v = jnp.dot(xn, vwt_ref[...], preferred_element_type=jnp.float32)
    cos = cos_ref[...]
    sin = sin_ref[...]

    def rope(u):
        ur = jnp.concatenate([-u[:, HALF:], u[:, :HALF]], axis=1)
        return u * cos + ur * sin

    rows = lax.broadcasted_iota(jnp.int32, (T, T), 0)
    cols = lax.broadcasted_iota(jnp.int32, (T, T), 1)
    causal = rows >= cols
    scale = 1.0 / math.sqrt(HEAD_DIM)
    ys = []
    kh_c = [rope(k[:, h * HEAD_DIM:(h + 1) * HEAD_DIM]).astype(jnp.bfloat16)
            for h in range(N_KV_HEAD)]
    for h in range(N_HEAD):
        qh = rope(q[:, h * HEAD_DIM:(h + 1) * HEAD_DIM]).astype(jnp.bfloat16)
        kh = kh_c[h // 2]
        vh = v[:, (h // 2) * HEAD_DIM:(h // 2 + 1) * HEAD_DIM].astype(jnp.bfloat16)
        s = lax.dot_general(qh, kh, (((1,), (1,)), ((), ())),
                            preferred_element_type=jnp.float32) * scale
        s = jnp.where(causal, s, -1e30)
        m = jnp.max(s, axis=1, keepdims=True)
        p = jnp.exp(s - m)
        a = (p / jnp.sum(p, axis=1, keepdims=True)).astype(jnp.bfloat16)
        ys.append(jnp.dot(a, vh, preferred_element_type=jnp.float32))
    y = jnp.concatenate(ys, axis=1).astype(jnp.bfloat16)
    return jnp.dot(y, owt_ref[...], preferred_element_type=jnp.float32) + xb


def _attn_body(x_ref, w1_ref, qwt_ref, kwt_ref, vwt_ref, owt_ref, cos_ref,
               sin_ref, x1_ref):
    for j in range(BB):
        x1_ref[j] = _attn_one(x_ref[j], w1_ref, qwt_ref, kwt_ref, vwt_ref,
                              owt_ref, cos_ref, sin_ref)


def _attention(x, ln1_w, qW, kW, vW, oW):
    cos, sin = _rope_tables()
    return pl.pallas_call(
        _attn_body,
        grid=(B // BB,),
        in_specs=[
            pl.BlockSpec((BB, T, C), lambda b: (b, 0, 0)),
            pl.BlockSpec((1, C), lambda b: (0, 0)),
            pl.BlockSpec((C, C), lambda b: (0, 0)),
            pl.BlockSpec((C, N_KV_HEAD * HEAD_DIM), lambda b: (0, 0)),
            pl.BlockSpec((C, N_KV_HEAD * HEAD_DIM), lambda b: (0, 0)),
            pl.BlockSpec((C, C), lambda b: (0, 0)),
            pl.BlockSpec((T, HEAD_DIM), lambda b: (0, 0)),
            pl.BlockSpec((T, HEAD_DIM), lambda b: (0, 0)),
        ],
        out_specs=pl.BlockSpec((BB, T, C), lambda b: (b, 0, 0)),
        out_shape=jax.ShapeDtypeStruct((B, T, C), jnp.float32),
    )(x, ln1_w.reshape(1, C), qW.T.astype(jnp.bfloat16),
      kW.T.astype(jnp.bfloat16), vW.T.astype(jnp.bfloat16),
      oW.T.astype(jnp.bfloat16), cos, sin)


def _router_body(x_ref, w_ref, rwt_ref, tril_ref, x2_ref, mf_ref, cnt_ref,
                 run_ref):
    t = pl.program_id(0)

    @pl.when(t == 0)
    def _():
        run_ref[...] = jnp.zeros((1, E), jnp.float32)

    xb = x_ref[...]
    xn = xb * lax.rsqrt(jnp.mean(xb * xb, axis=-1, keepdims=True) + 1e-6)
    xn = xn * w_ref[...]
    x2_ref[...] = xn

    logits = jnp.dot(xn, rwt_ref[...], preferred_element_type=jnp.float32)
    lm = jnp.max(logits, axis=1, keepdims=True)
    ex = jnp.exp(logits - lm)
    probs = ex / jnp.sum(ex, axis=1, keepdims=True)

    lane = lax.broadcasted_iota(jnp.int32, (TILE, E), 1)
    p0 = jnp.max(probs, axis=1, keepdims=True)
    i0 = jnp.min(jnp.where(probs == p0, lane, E), axis=1, keepdims=True)
    masked = jnp.where(lane == i0, -1.0, probs)
    p1 = jnp.max(masked, axis=1, keepdims=True)
    i1 = jnp.min(jnp.where(masked == p1, lane, E), axis=1, keepdims=True)
    ssum = p0 + p1 + 1e-9
    g0 = p0 / ssum
    g1 = p1 / ssum

    oh0 = (lane == i0).astype(jnp.float32)
    oh1 = (lane == i1).astype(jnp.float32)
    oh = oh0 + oh1
    prefix = jnp.dot(tril_ref[...], oh, preferred_element_type=jnp.float32)
    run = run_ref[...]
    r0 = jnp.sum((prefix + run) * oh0, axis=1, keepdims=True)
    r1 = jnp.sum((prefix + run) * oh1, axis=1, keepdims=True)
    newrun = run + jnp.sum(oh, axis=0, keepdims=True)
    run_ref[...] = newrun
    cnt_ref[...] = newrun

    mf_ref[0] = jnp.concatenate(
        [i0.astype(jnp.float32), i1.astype(jnp.float32), r0, r1, g0, g1,
         jnp.zeros((TILE, 2), jnp.float32)], axis=1)


def _router(x1f, ln2_w, routerW):
    tril = np.tril(np.ones((TILE, TILE), np.float32), k=-1)
    return pl.pallas_call(
        _router_body,
        grid=(NTOK_TILES,),
        in_specs=[
            pl.BlockSpec((TILE, C), lambda t: (t, 0)),
            pl.BlockSpec((1, C), lambda t: (0, 0)),
            pl.BlockSpec((C, E), lambda t: (0, 0)),
            pl.BlockSpec((TILE, TILE), lambda t: (0, 0)),
        ],
        out_specs=[
            pl.BlockSpec((TILE, C), lambda t: (t, 0)),
            pl.BlockSpec((1, TILE, E), lambda t: (t, 0, 0)),
            pl.BlockSpec((1, E), lambda t: (0, 0)),
        ],
        out_shape=[
            jax.ShapeDtypeStruct((N, C), jnp.float32),
            jax.ShapeDtypeStruct((NTOK_TILES, TILE, E), jnp.float32),
            jax.ShapeDtypeStruct((1, E), jnp.float32),
        ],
        scratch_shapes=[pltpu.VMEM((1, E), jnp.float32)],
    )(x1f, ln2_w.reshape(1, C), routerW.T, jnp.asarray(tril))


def _dest_body(mf_ref, cnt_ref, tril8_ref, dd0_ref, dd1_ref, texp_ref):
    cnt = cnt_ref[...]
    padded = jnp.ceil(cnt / TILE) * TILE
    offs = jnp.dot(padded, tril8_ref[...], preferred_element_type=jnp.float32)
    ends = (offs + padded) / TILE
    tt = lax.broadcasted_iota(jnp.int32, (1, 128), 1).astype(jnp.float32)
    acc = jnp.zeros((1, 128), jnp.float32)
    for e in range(E):
        acc = acc + (tt >= ends[0, e]).astype(jnp.float32)
    texp_ref[...] = jnp.minimum(acc, float(E - 1))

    e0 = mf_ref[:, :, 0]
    e1 = mf_ref[:, :, 1]
    d0 = mf_ref[:, :, 2]
    d1 = mf_ref[:, :, 3]
    for e in range(E):
        d0 = d0 + jnp.where(e0 == float(e), offs[0, e], 0.0)
        d1 = d1 + jnp.where(e1 == float(e), offs[0, e], 0.0)
    dd0_ref[...] = d0
    dd1_ref[...] = d1


def _dest(mf, cnt):
    tril8 = np.triu(np.ones((E, E), np.float32), k=1)
    return pl.pallas_call(
        _dest_body,
        out_shape=[
            jax.ShapeDtypeStruct((NTOK_TILES, TILE), jnp.float32),
            jax.ShapeDtypeStruct((NTOK_TILES, TILE), jnp.float32),
            jax.ShapeDtypeStruct((1, 128), jnp.float32),
        ],
    )(mf, cnt, jnp.asarray(tril8))


CH = 128
NCH = TPW // CH


def _dispatch_body(x2_hbm, d0_hbm, d1_hbm, buf_hbm, *refs):
    rows = refs[0:NCH]
    dv0 = refs[NCH:2 * NCH]
    dv1 = refs[2 * NCH:3 * NCH]
    lsem, ssem = refs[3 * NCH], refs[3 * NCH + 1]
    wid = lax.axis_index("s") * 2 + lax.axis_index("c")
    loads = []
    for ci in range(NCH):
        base = wid * TPW + ci * CH
        loads.append(pltpu.async_copy(x2_hbm.at[pl.ds(base, CH)], rows[ci], lsem))
        loads.append(pltpu.async_copy(d0_hbm.at[pl.ds(base, CH)], dv0[ci], lsem))
        loads.append(pltpu.async_copy(d1_hbm.at[pl.ds(base, CH)], dv1[ci], lsem))
    for c in loads:
        c.wait()
    scs = []
    for ci in range(NCH):
        scs.append(pltpu.async_copy(rows[ci], buf_hbm.at[dv0[ci]], ssem))
        scs.append(pltpu.async_copy(rows[ci], buf_hbm.at[dv1[ci]], ssem))
    for c in scs:
        c.wait()


def _dispatch(x2, d0, d1):
    mesh = plsc.VectorSubcoreMesh(core_axis_name="c", subcore_axis_name="s")
    f = pl.kernel(
        _dispatch_body,
        mesh=mesh,
        out_type=jax.ShapeDtypeStruct((PBUF, C), jnp.float32),
        scratch_types=(
            [pltpu.VMEM((CH, C), jnp.float32) for _ in range(NCH)]
            + [pltpu.VMEM((CH,), jnp.int32) for _ in range(2 * NCH)]
            + [pltpu.SemaphoreType.DMA, pltpu.SemaphoreType.DMA]
        ),
    )
    return f(x2, d0, d1)


def _gmm_body(texp_ref, xb_ref, w1_ref, w3_ref, w2_ref, y_ref):
    xb = xb_ref[...].astype(jnp.bfloat16)
    h1 = lax.dot_general(xb, w1_ref[0], (((1,), (1,)), ((), ())),
                         preferred_element_type=jnp.float32)
    h3 = lax.dot_general(xb, w3_ref[0], (((1,), (1,)), ((), ())),
                         preferred_element_type=jnp.float32)
    h = (h1 * (1.0 / (1.0 + jnp.exp(-h1))) * h3).astype(jnp.bfloat16)
    y_ref[...] = lax.dot_general(h, w2_ref[0], (((1,), (1,)), ((), ())),
                                 preferred_element_type=jnp.float32)


def _grouped_mlp(texp, buf, W1p, W3p, W2p):
    grid_spec = pltpu.PrefetchScalarGridSpec(
        num_scalar_prefetch=1,
        grid=(NTILES,),
        in_specs=[
            pl.BlockSpec((TILE, C), lambda t, s: (t, 0)),
            pl.BlockSpec((1, HPAD, C), lambda t, s: (s[t], 0, 0)),
            pl.BlockSpec((1, HPAD, C), lambda t, s: (s[t], 0, 0)),
            pl.BlockSpec((1, C, HPAD), lambda t, s: (s[t], 0, 0)),
        ],
        out_specs=pl.BlockSpec((TILE, C), lambda t, s: (t, 0)),
    )
    return pl.pallas_call(
        _gmm_body,
        grid_spec=grid_spec,
        out_shape=jax.ShapeDtypeStruct((PBUF, C), jnp.float32),
    )(texp, buf, W1p, W3p, W2p)


CCH = 64
NCHG = TPW // CCH


def _gather_body(y_hbm, d0_hbm, d1_hbm, r0_hbm, r1_hbm, *refs):
    ry0 = refs[0:2]
    ry1 = refs[2:4]
    dv0 = refs[4:6]
    dv1 = refs[6:8]
    gsem, ssem = refs[8], refs[9]
    wid = lax.axis_index("s") * 2 + lax.axis_index("c")
    stores = [None, None]
    for ci in range(NCHG):
        s = ci % 2
        if stores[s] is not None:
            stores[s][0].wait()
            stores[s][1].wait()
        base = wid * TPW + ci * CCH
        pltpu.sync_copy(d0_hbm.at[pl.ds(base, CCH)], dv0[s])
        pltpu.sync_copy(d1_hbm.at[pl.ds(base, CCH)], dv1[s])
        c0 = pltpu.async_copy(y_hbm.at[dv0[s]], ry0[s], gsem)
        c1 = pltpu.async_copy(y_hbm.at[dv1[s]], ry1[s], gsem)
        c0.wait()
        c1.wait()
        stores[s] = (
            pltpu.async_copy(ry0[s], r0_hbm.at[pl.ds(base, CCH)], ssem),
            pltpu.async_copy(ry1[s], r1_hbm.at[pl.ds(base, CCH)], ssem),
        )
    for st in stores:
        st[0].wait()
        st[1].wait()


def _gather2(y, d0, d1):
    mesh = plsc.VectorSubcoreMesh(core_axis_name="c", subcore_axis_name="s")
    f = pl.kernel(
        _gather_body,
        mesh=mesh,
        out_type=[
            jax.ShapeDtypeStruct((N, C), jnp.float32),
            jax.ShapeDtypeStruct((N, C), jnp.float32),
        ],
        scratch_types=(
            [pltpu.VMEM((CCH, C), jnp.float32) for _ in range(4)]
            + [pltpu.VMEM((CCH,), jnp.int32) for _ in range(4)]
            + [pltpu.SemaphoreType.DMA] * 2
        ),
    )
    return f(y, d0, d1)


def _epilogue_body(x1_ref, ry0_ref, ry1_ref, g0_ref, g1_ref, out_ref):
    out_ref[...] = (x1_ref[...] + g0_ref[...] * ry0_ref[...]
                    + g1_ref[...] * ry1_ref[...])


def _epilogue(x1f, ry0, ry1, g0, g1):
    return pl.pallas_call(
        _epilogue_body,
        grid=(NTOK_TILES,),
        in_specs=[
            pl.BlockSpec((TILE, C), lambda t: (t, 0)),
            pl.BlockSpec((TILE, C), lambda t: (t, 0)),
            pl.BlockSpec((TILE, C), lambda t: (t, 0)),
            pl.BlockSpec((TILE, 1), lambda t: (t, 0)),
            pl.BlockSpec((TILE, 1), lambda t: (t, 0)),
        ],
        out_specs=pl.BlockSpec((TILE, C), lambda t: (t, 0)),
        out_shape=jax.ShapeDtypeStruct((N, C), jnp.float32),
    )(x1f, ry0, ry1, g0, g1)


def kernel(x, ln1_w, qW, kW, vW, oW, ln2_w, routerW, W1, W2, W3):
    x1 = _attention(x, ln1_w, qW, kW, vW, oW)
    x1f = x1.reshape(N, C)
    x2, mf, cnt = _router(x1f, ln2_w, routerW)
    dd0, dd1, texpf = _dest(mf, cnt)
    texp = texpf[0, :NTILES].astype(jnp.int32)

    d0 = dd0.reshape(N).astype(jnp.int32)
    d1 = dd1.reshape(N).astype(jnp.int32)
    g0 = mf[:, :, 4].reshape(N, 1)
    g1 = mf[:, :, 5].reshape(N, 1)

    buf = _dispatch(x2, d0, d1)

    W1p = jnp.pad(W1.astype(jnp.bfloat16), ((0, 0), (0, HPAD - HID), (0, 0)))
    W3p = jnp.pad(W3.astype(jnp.bfloat16), ((0, 0), (0, HPAD - HID), (0, 0)))
    W2p = jnp.pad(W2.astype(jnp.bfloat16), ((0, 0), (0, 0), (0, HPAD - HID)))
    y = _grouped_mlp(texp, buf, W1p, W3p, W2p)

    ry0, ry1 = _gather2(y, d0, d1)
    out = _epilogue(x1f, ry0, ry1, g0, g1)
    return out.reshape(B, T, C)
